# baseline (speedup 1.0000x reference)
import functools

import jax
import jax.numpy as jnp
from jax import lax
from jax.experimental import pallas as pl
from jax.experimental.pallas import tpu as pltpu
from jax.experimental.pallas import tpu_sc as plsc

B = 1024
BH = B // 2
CTX = 50
D = 16
V = 100001

BV = 1024
NV = (V + BV - 1) // BV
VP = NV * BV

GRP = 16


def _sc_info():
    try:
        info = plsc.get_sparse_core_info()
        return info.num_cores, info.num_subcores
    except Exception:
        return 2, 16


def _make_embed_sum(nc, ns):
    nw = nc * ns
    halves = nw // D
    bpw = B // halves
    mesh = plsc.VectorSubcoreMesh(
        core_axis_name="c", subcore_axis_name="s",
        num_cores=nc, num_subcores=ns)

    @functools.partial(
        pl.kernel,
        out_type=jax.ShapeDtypeStruct((D, B), jnp.float32),
        mesh=mesh,
        scratch_types=[
            pltpu.VMEM((V,), jnp.float32),
            pltpu.VMEM((CTX, bpw), jnp.int32),
            pltpu.VMEM((1, bpw), jnp.float32),
        ],
        compiler_params=pltpu.CompilerParams(
            use_tc_tiling_on_sc=False, needs_layout_passes=False),
    )
    def embed_sum(idx_hbm, table_t_hbm, out_hbm, row_v, idx_v, acc_v):
        wid = lax.axis_index("s") * nc + lax.axis_index("c")
        d = wid // halves
        boff = (wid % halves) * bpw
        pltpu.sync_copy(table_t_hbm.at[d], row_v)
        pltpu.sync_copy(idx_hbm.at[:, pl.ds(boff, bpw)], idx_v)

        def outer(k, _):
            def inner(c, acc):
                iv = idx_v[c, pl.ds(k * GRP, GRP)]
                return acc + plsc.load_gather(row_v, [iv])
            acc_v[0, pl.ds(k * GRP, GRP)] = lax.fori_loop(
                0, CTX, inner, jnp.zeros((GRP,), jnp.float32))
            return 0

        lax.fori_loop(0, bpw // GRP, outer, 0)
        pltpu.sync_copy(acc_v, out_hbm.at[pl.ds(d, 1), pl.ds(boff, bpw)])

    return embed_sum


def _fused_body(st_ref, w_ref, o_ref, m_scr, s_scr, c_scr):
    p = pl.program_id(0)
    i = pl.program_id(1)

    def _logits(h):
        return lax.dot_general(
            w_ref[...], st_ref[:, h * BH:(h + 1) * BH],
            (((0,), (0,)), ((), ())),
            preferred_element_type=jnp.float32)

    def _stats_step(h):
        sl = slice(h * BH, (h + 1) * BH)
        logits = _logits(h)
        m_prev = m_scr[:, sl]
        m_new = jnp.maximum(m_prev, jnp.max(logits, axis=0, keepdims=True))
        e = jnp.exp(logits - m_new)
        bsum = lax.dot_general(
            jnp.ones((1, BV), jnp.float32), e, (((1,), (0,)), ((), ())),
            preferred_element_type=jnp.float32)
        s_scr[:, sl] = s_scr[:, sl] * jnp.exp(m_prev - m_new) + bsum
        m_scr[:, sl] = m_new

    def _write_step(h):
        sl = slice(h * BH, (h + 1) * BH)
        o_ref[...] = jnp.exp(_logits(h) - c_scr[:, sl])

    @pl.when(jnp.logical_and(p == 0, i == 0))
    def _():
        m_scr[...] = jnp.full(m_scr.shape, -jnp.inf, jnp.float32)
        s_scr[...] = jnp.zeros(s_scr.shape, jnp.float32)

    @pl.when(p == 0)
    def _():
        _stats_step(0)

    @pl.when(p == 1)
    def _():
        @pl.when(i == 0)
        def _():
            c_scr[:, 0:BH] = m_scr[:, 0:BH] + jnp.log(s_scr[:, 0:BH])

        _write_step(0)
        _stats_step(1)

    @pl.when(p == 2)
    def _():
        @pl.when(i == 0)
        def _():
            c_scr[:, BH:B] = m_scr[:, BH:B] + jnp.log(s_scr[:, BH:B])

        _write_step(1)


def _out_index(p, i):
    return (jnp.where(p == 0, 0, i), jnp.maximum(p - 1, 0))


def _softmax_fused(st_aug, wt_aug):
    return pl.pallas_call(
        _fused_body,
        grid=(3, NV),
        in_specs=[
            pl.BlockSpec((D + 1, B), lambda p, i: (0, 0)),
            pl.BlockSpec((D + 1, BV), lambda p, i: (0, i)),
        ],
        out_specs=pl.BlockSpec((BV, BH), _out_index),
        out_shape=jax.ShapeDtypeStruct((V, B), jnp.float32),
        scratch_shapes=[
            pltpu.VMEM((1, B), jnp.float32),
            pltpu.VMEM((1, B), jnp.float32),
            pltpu.VMEM((1, B), jnp.float32),
        ],
        compiler_params=pltpu.CompilerParams(
            dimension_semantics=("arbitrary", "arbitrary")),
    )(st_aug, wt_aug)


def kernel(inputs, table, W, b):
    nc, ns = _sc_info()
    summed_t = _make_embed_sum(nc, ns)(inputs.T, table.T)
    st_aug = jnp.concatenate([summed_t, jnp.ones((1, B), jnp.float32)], axis=0)
    wt_pad = jnp.pad(W.T, ((0, 0), (0, VP - V)))
    b_pad = jnp.pad(b, (0, VP - V), constant_values=-1e30)
    wt_aug = jnp.concatenate([wt_pad, b_pad[None, :]], axis=0)
    out_t = _softmax_fused(st_aug, wt_aug)
    return out_t.T


# speedup vs baseline: 1.2447x; 1.2447x over previous
import functools

import jax
import jax.numpy as jnp
from jax import lax
from jax.experimental import pallas as pl
from jax.experimental.pallas import tpu as pltpu
from jax.experimental.pallas import tpu_sc as plsc

B = 1024
CTX = 50
D = 16
V = 100001

BV = 2048
NV = (V + BV - 1) // BV
VP = NV * BV

GRP = 16


def _sc_info():
    try:
        info = plsc.get_sparse_core_info()
        return info.num_cores, info.num_subcores
    except Exception:
        return 2, 16


def _make_embed_sum(nc, ns):
    nw = nc * ns
    halves = nw // D
    bpw = B // halves
    mesh = plsc.VectorSubcoreMesh(
        core_axis_name="c", subcore_axis_name="s",
        num_cores=nc, num_subcores=ns)

    @functools.partial(
        pl.kernel,
        out_type=jax.ShapeDtypeStruct((D, B), jnp.float32),
        mesh=mesh,
        scratch_types=[
            pltpu.VMEM((V,), jnp.float32),
            pltpu.VMEM((CTX, bpw), jnp.int32),
            pltpu.VMEM((1, bpw), jnp.float32),
        ],
        compiler_params=pltpu.CompilerParams(
            use_tc_tiling_on_sc=False, needs_layout_passes=False),
    )
    def embed_sum(idx_hbm, table_t_hbm, out_hbm, row_v, idx_v, acc_v):
        wid = lax.axis_index("s") * nc + lax.axis_index("c")
        d = wid // halves
        boff = (wid % halves) * bpw
        pltpu.sync_copy(table_t_hbm.at[d], row_v)
        pltpu.sync_copy(idx_hbm.at[:, pl.ds(boff, bpw)], idx_v)

        def outer(k, _):
            def inner(c, acc):
                iv = idx_v[c, pl.ds(k * GRP, GRP)]
                return acc + plsc.load_gather(row_v, [iv])
            acc_v[0, pl.ds(k * GRP, GRP)] = lax.fori_loop(
                0, CTX, inner, jnp.zeros((GRP,), jnp.float32))
            return 0

        lax.fori_loop(0, bpw // GRP, outer, 0)
        pltpu.sync_copy(acc_v, out_hbm.at[pl.ds(d, 1), pl.ds(boff, bpw)])

    return embed_sum


def _block_logits_t(st_ref, w_ref):
    return lax.dot_general(
        w_ref[...], st_ref[...], (((0,), (0,)), ((), ())),
        preferred_element_type=jnp.float32)


def _stats_body(st_ref, w_ref, m_out, s_out, m_scr, s_scr):
    i = pl.program_id(0)
    logits = _block_logits_t(st_ref, w_ref)

    @pl.when(i == 0)
    def _():
        m_scr[...] = jnp.full(m_scr.shape, -jnp.inf, jnp.float32)
        s_scr[...] = jnp.zeros(s_scr.shape, jnp.float32)

    m_prev = m_scr[...]
    m_new = jnp.maximum(m_prev, jnp.max(logits, axis=0, keepdims=True))
    e = jnp.exp(logits - m_new)
    bsum = lax.dot_general(
        jnp.ones((1, BV), jnp.float32), e, (((1,), (0,)), ((), ())),
        preferred_element_type=jnp.float32)
    s_scr[...] = s_scr[...] * jnp.exp(m_prev - m_new) + bsum
    m_scr[...] = m_new

    @pl.when(i == NV - 1)
    def _():
        m_out[...] = m_scr[...]
        s_out[...] = s_scr[...]


def _out_body(st_ref, w_ref, c_ref, o_ref):
    logits = _block_logits_t(st_ref, w_ref)
    o_ref[...] = jnp.exp(logits - c_ref[...])


def _softmax_stats(st_aug, wt_aug):
    return pl.pallas_call(
        _stats_body,
        grid=(NV,),
        in_specs=[
            pl.BlockSpec((D + 1, B), lambda i: (0, 0)),
            pl.BlockSpec((D + 1, BV), lambda i: (0, i)),
        ],
        out_specs=[
            pl.BlockSpec((1, B), lambda i: (0, 0)),
            pl.BlockSpec((1, B), lambda i: (0, 0)),
        ],
        out_shape=[
            jax.ShapeDtypeStruct((1, B), jnp.float32),
            jax.ShapeDtypeStruct((1, B), jnp.float32),
        ],
        scratch_shapes=[
            pltpu.VMEM((1, B), jnp.float32),
            pltpu.VMEM((1, B), jnp.float32),
        ],
        compiler_params=pltpu.CompilerParams(
            dimension_semantics=("arbitrary",)),
    )(st_aug, wt_aug)


def _softmax_write(st_aug, wt_aug, c):
    return pl.pallas_call(
        _out_body,
        grid=(NV,),
        in_specs=[
            pl.BlockSpec((D + 1, B), lambda i: (0, 0)),
            pl.BlockSpec((D + 1, BV), lambda i: (0, i)),
            pl.BlockSpec((1, B), lambda i: (0, 0)),
        ],
        out_specs=pl.BlockSpec((BV, B), lambda i: (i, 0)),
        out_shape=jax.ShapeDtypeStruct((V, B), jnp.float32),
        compiler_params=pltpu.CompilerParams(
            dimension_semantics=("arbitrary",)),
    )(st_aug, wt_aug, c)


def kernel(inputs, table, W, b):
    nc, ns = _sc_info()
    summed_t = _make_embed_sum(nc, ns)(inputs.T, table.T)
    st_aug = jnp.concatenate([summed_t, jnp.ones((1, B), jnp.float32)], axis=0)
    wt_pad = jnp.pad(W.T, ((0, 0), (0, VP - V)))
    b_pad = jnp.pad(b, (0, VP - V), constant_values=-1e30)
    wt_aug = jnp.concatenate([wt_pad, b_pad[None, :]], axis=0)
    m, s = _softmax_stats(st_aug, wt_aug)
    c = m + jnp.log(s)
    out_t = _softmax_write(st_aug, wt_aug, c)
    return out_t.T


# speedup vs baseline: 1.2691x; 1.0196x over previous
import functools

import jax
import jax.numpy as jnp
from jax import lax
from jax.experimental import pallas as pl
from jax.experimental.pallas import tpu as pltpu
from jax.experimental.pallas import tpu_sc as plsc

B = 1024
CTX = 50
D = 16
V = 100001

BV = 4096
NV = (V + BV - 1) // BV
VP = NV * BV

GRP = 16


def _sc_info():
    try:
        info = plsc.get_sparse_core_info()
        return info.num_cores, info.num_subcores
    except Exception:
        return 2, 16


def _make_embed_sum(nc, ns):
    nw = nc * ns
    halves = nw // D
    bpw = B // halves
    mesh = plsc.VectorSubcoreMesh(
        core_axis_name="c", subcore_axis_name="s",
        num_cores=nc, num_subcores=ns)

    @functools.partial(
        pl.kernel,
        out_type=jax.ShapeDtypeStruct((D, B), jnp.float32),
        mesh=mesh,
        scratch_types=[
            pltpu.VMEM((V,), jnp.float32),
            pltpu.VMEM((CTX, bpw), jnp.int32),
            pltpu.VMEM((1, bpw), jnp.float32),
        ],
        compiler_params=pltpu.CompilerParams(
            use_tc_tiling_on_sc=False, needs_layout_passes=False),
    )
    def embed_sum(idx_hbm, table_t_hbm, out_hbm, row_v, idx_v, acc_v):
        wid = lax.axis_index("s") * nc + lax.axis_index("c")
        d = wid // halves
        boff = (wid % halves) * bpw
        pltpu.sync_copy(table_t_hbm.at[d], row_v)
        pltpu.sync_copy(idx_hbm.at[:, pl.ds(boff, bpw)], idx_v)

        def outer(k, _):
            def inner(c, acc):
                iv = idx_v[c, pl.ds(k * GRP, GRP)]
                return acc + plsc.load_gather(row_v, [iv])
            acc_v[0, pl.ds(k * GRP, GRP)] = lax.fori_loop(
                0, CTX, inner, jnp.zeros((GRP,), jnp.float32))
            return 0

        lax.fori_loop(0, bpw // GRP, outer, 0)
        pltpu.sync_copy(acc_v, out_hbm.at[pl.ds(d, 1), pl.ds(boff, bpw)])

    return embed_sum


def _block_logits_t(st_ref, w_ref):
    return lax.dot_general(
        w_ref[...], st_ref[...], (((0,), (0,)), ((), ())),
        preferred_element_type=jnp.float32)


def _stats_body(st_ref, w_ref, m_out, s_out, m_scr, s_scr):
    i = pl.program_id(0)
    logits = _block_logits_t(st_ref, w_ref)

    @pl.when(i == 0)
    def _():
        m_scr[...] = jnp.full(m_scr.shape, -jnp.inf, jnp.float32)
        s_scr[...] = jnp.zeros(s_scr.shape, jnp.float32)

    m_prev = m_scr[...]
    m_new = jnp.maximum(m_prev, jnp.max(logits, axis=0, keepdims=True))
    e = jnp.exp(logits - m_new)
    bsum = lax.dot_general(
        jnp.ones((1, BV), jnp.float32), e, (((1,), (0,)), ((), ())),
        preferred_element_type=jnp.float32)
    s_scr[...] = s_scr[...] * jnp.exp(m_prev - m_new) + bsum
    m_scr[...] = m_new

    @pl.when(i == NV - 1)
    def _():
        m_out[...] = m_scr[...]
        s_out[...] = s_scr[...]


def _out_body(st_ref, w_ref, c_ref, o_ref):
    logits = _block_logits_t(st_ref, w_ref)
    o_ref[...] = jnp.exp(logits - c_ref[...])


def _softmax_stats(st_aug, wt_aug):
    return pl.pallas_call(
        _stats_body,
        grid=(NV,),
        in_specs=[
            pl.BlockSpec((D + 1, B), lambda i: (0, 0)),
            pl.BlockSpec((D + 1, BV), lambda i: (0, i)),
        ],
        out_specs=[
            pl.BlockSpec((1, B), lambda i: (0, 0)),
            pl.BlockSpec((1, B), lambda i: (0, 0)),
        ],
        out_shape=[
            jax.ShapeDtypeStruct((1, B), jnp.float32),
            jax.ShapeDtypeStruct((1, B), jnp.float32),
        ],
        scratch_shapes=[
            pltpu.VMEM((1, B), jnp.float32),
            pltpu.VMEM((1, B), jnp.float32),
        ],
        compiler_params=pltpu.CompilerParams(
            dimension_semantics=("arbitrary",)),
    )(st_aug, wt_aug)


def _softmax_write(st_aug, wt_aug, c):
    return pl.pallas_call(
        _out_body,
        grid=(NV,),
        in_specs=[
            pl.BlockSpec((D + 1, B), lambda i: (0, 0)),
            pl.BlockSpec((D + 1, BV), lambda i: (0, i)),
            pl.BlockSpec((1, B), lambda i: (0, 0)),
        ],
        out_specs=pl.BlockSpec((BV, B), lambda i: (i, 0)),
        out_shape=jax.ShapeDtypeStruct((V, B), jnp.float32),
        compiler_params=pltpu.CompilerParams(
            dimension_semantics=("arbitrary",)),
    )(st_aug, wt_aug, c)


def kernel(inputs, table, W, b):
    nc, ns = _sc_info()
    summed_t = _make_embed_sum(nc, ns)(inputs.T, table.T)
    st_aug = jnp.concatenate([summed_t, jnp.ones((1, B), jnp.float32)], axis=0)
    wt_pad = jnp.pad(W.T, ((0, 0), (0, VP - V)))
    b_pad = jnp.pad(b, (0, VP - V), constant_values=-1e30)
    wt_aug = jnp.concatenate([wt_pad, b_pad[None, :]], axis=0)
    m, s = _softmax_stats(st_aug, wt_aug)
    c = m + jnp.log(s)
    out_t = _softmax_write(st_aug, wt_aug, c)
    return out_t.T


# speedup vs baseline: 1.2700x; 1.0007x over previous
import functools

import jax
import jax.numpy as jnp
from jax import lax
from jax.experimental import pallas as pl
from jax.experimental.pallas import tpu as pltpu
from jax.experimental.pallas import tpu_sc as plsc

B = 1024
CTX = 50
D = 16
V = 100001

BV = 5120
NV = (V + BV - 1) // BV
VP = NV * BV

GRP = 16


def _sc_info():
    try:
        info = plsc.get_sparse_core_info()
        return info.num_cores, info.num_subcores
    except Exception:
        return 2, 16


def _make_embed_sum(nc, ns):
    nw = nc * ns
    halves = nw // D
    bpw = B // halves
    mesh = plsc.VectorSubcoreMesh(
        core_axis_name="c", subcore_axis_name="s",
        num_cores=nc, num_subcores=ns)

    @functools.partial(
        pl.kernel,
        out_type=jax.ShapeDtypeStruct((D, B), jnp.float32),
        mesh=mesh,
        scratch_types=[
            pltpu.VMEM((V,), jnp.float32),
            pltpu.VMEM((CTX, bpw), jnp.int32),
            pltpu.VMEM((1, bpw), jnp.float32),
        ],
        compiler_params=pltpu.CompilerParams(
            use_tc_tiling_on_sc=False, needs_layout_passes=False),
    )
    def embed_sum(idx_hbm, table_t_hbm, out_hbm, row_v, idx_v, acc_v):
        wid = lax.axis_index("s") * nc + lax.axis_index("c")
        d = wid // halves
        boff = (wid % halves) * bpw
        pltpu.sync_copy(table_t_hbm.at[d], row_v)
        pltpu.sync_copy(idx_hbm.at[:, pl.ds(boff, bpw)], idx_v)

        def outer(k, _):
            def inner(c, acc):
                iv = idx_v[c, pl.ds(k * GRP, GRP)]
                return acc + plsc.load_gather(row_v, [iv])
            acc_v[0, pl.ds(k * GRP, GRP)] = lax.fori_loop(
                0, CTX, inner, jnp.zeros((GRP,), jnp.float32))
            return 0

        lax.fori_loop(0, bpw // GRP, outer, 0)
        pltpu.sync_copy(acc_v, out_hbm.at[pl.ds(d, 1), pl.ds(boff, bpw)])

    return embed_sum


def _block_logits_t(st_ref, w_ref):
    return lax.dot_general(
        w_ref[...], st_ref[...], (((0,), (0,)), ((), ())),
        preferred_element_type=jnp.float32)


def _stats_body(st_ref, w_ref, m_out, s_out, m_scr, s_scr):
    i = pl.program_id(0)
    logits = _block_logits_t(st_ref, w_ref)

    @pl.when(i == 0)
    def _():
        m_scr[...] = jnp.full(m_scr.shape, -jnp.inf, jnp.float32)
        s_scr[...] = jnp.zeros(s_scr.shape, jnp.float32)

    m_prev = m_scr[...]
    m_new = jnp.maximum(m_prev, jnp.max(logits, axis=0, keepdims=True))
    e = jnp.exp(logits - m_new)
    bsum = lax.dot_general(
        jnp.ones((1, BV), jnp.float32), e, (((1,), (0,)), ((), ())),
        preferred_element_type=jnp.float32)
    s_scr[...] = s_scr[...] * jnp.exp(m_prev - m_new) + bsum
    m_scr[...] = m_new

    @pl.when(i == NV - 1)
    def _():
        m_out[...] = m_scr[...]
        s_out[...] = s_scr[...]


def _out_body(st_ref, w_ref, c_ref, o_ref):
    logits = _block_logits_t(st_ref, w_ref)
    o_ref[...] = jnp.exp(logits - c_ref[...])


def _softmax_stats(st_aug, wt_aug):
    return pl.pallas_call(
        _stats_body,
        grid=(NV,),
        in_specs=[
            pl.BlockSpec((D + 1, B), lambda i: (0, 0)),
            pl.BlockSpec((D + 1, BV), lambda i: (0, i)),
        ],
        out_specs=[
            pl.BlockSpec((1, B), lambda i: (0, 0)),
            pl.BlockSpec((1, B), lambda i: (0, 0)),
        ],
        out_shape=[
            jax.ShapeDtypeStruct((1, B), jnp.float32),
            jax.ShapeDtypeStruct((1, B), jnp.float32),
        ],
        scratch_shapes=[
            pltpu.VMEM((1, B), jnp.float32),
            pltpu.VMEM((1, B), jnp.float32),
        ],
        compiler_params=pltpu.CompilerParams(
            dimension_semantics=("arbitrary",)),
    )(st_aug, wt_aug)


def _softmax_write(st_aug, wt_aug, c):
    return pl.pallas_call(
        _out_body,
        grid=(NV,),
        in_specs=[
            pl.BlockSpec((D + 1, B), lambda i: (0, 0)),
            pl.BlockSpec((D + 1, BV), lambda i: (0, i)),
            pl.BlockSpec((1, B), lambda i: (0, 0)),
        ],
        out_specs=pl.BlockSpec((BV, B), lambda i: (i, 0)),
        out_shape=jax.ShapeDtypeStruct((V, B), jnp.float32),
        compiler_params=pltpu.CompilerParams(
            dimension_semantics=("arbitrary",)),
    )(st_aug, wt_aug, c)


def kernel(inputs, table, W, b):
    nc, ns = _sc_info()
    summed_t = _make_embed_sum(nc, ns)(inputs.T, table.T)
    st_aug = jnp.concatenate([summed_t, jnp.ones((1, B), jnp.float32)], axis=0)
    wt_pad = jnp.pad(W.T, ((0, 0), (0, VP - V)))
    b_pad = jnp.pad(b, (0, VP - V), constant_values=-1e30)
    wt_aug = jnp.concatenate([wt_pad, b_pad[None, :]], axis=0)
    m, s = _softmax_stats(st_aug, wt_aug)
    c = m + jnp.log(s)
    out_t = _softmax_write(st_aug, wt_aug, c)
    return out_t.T


# speedup vs baseline: 1.2816x; 1.0092x over previous
import functools

import jax
import jax.numpy as jnp
from jax import lax
from jax.experimental import pallas as pl
from jax.experimental.pallas import tpu as pltpu
from jax.experimental.pallas import tpu_sc as plsc

B = 1024
CTX = 50
D = 16
V = 100001

BV = 6272
NV = (V + BV - 1) // BV
VP = NV * BV

GRP = 16


def _sc_info():
    try:
        info = plsc.get_sparse_core_info()
        return info.num_cores, info.num_subcores
    except Exception:
        return 2, 16


def _make_embed_sum(nc, ns):
    nw = nc * ns
    halves = nw // D
    bpw = B // halves
    mesh = plsc.VectorSubcoreMesh(
        core_axis_name="c", subcore_axis_name="s",
        num_cores=nc, num_subcores=ns)

    @functools.partial(
        pl.kernel,
        out_type=jax.ShapeDtypeStruct((D, B), jnp.float32),
        mesh=mesh,
        scratch_types=[
            pltpu.VMEM((V,), jnp.float32),
            pltpu.VMEM((CTX, bpw), jnp.int32),
            pltpu.VMEM((1, bpw), jnp.float32),
        ],
        compiler_params=pltpu.CompilerParams(
            use_tc_tiling_on_sc=False, needs_layout_passes=False),
    )
    def embed_sum(idx_hbm, table_t_hbm, out_hbm, row_v, idx_v, acc_v):
        wid = lax.axis_index("s") * nc + lax.axis_index("c")
        d = wid // halves
        boff = (wid % halves) * bpw
        pltpu.sync_copy(table_t_hbm.at[d], row_v)
        pltpu.sync_copy(idx_hbm.at[:, pl.ds(boff, bpw)], idx_v)

        def outer(k, _):
            def inner(c, acc):
                iv = idx_v[c, pl.ds(k * GRP, GRP)]
                return acc + plsc.load_gather(row_v, [iv])
            acc_v[0, pl.ds(k * GRP, GRP)] = lax.fori_loop(
                0, CTX, inner, jnp.zeros((GRP,), jnp.float32))
            return 0

        lax.fori_loop(0, bpw // GRP, outer, 0)
        pltpu.sync_copy(acc_v, out_hbm.at[pl.ds(d, 1), pl.ds(boff, bpw)])

    return embed_sum


def _block_logits_t(st_ref, w_ref):
    return lax.dot_general(
        w_ref[...], st_ref[...], (((0,), (0,)), ((), ())),
        preferred_element_type=jnp.float32)


def _stats_body(st_ref, w_ref, m_out, s_out, m_scr, s_scr):
    i = pl.program_id(0)
    logits = _block_logits_t(st_ref, w_ref)

    @pl.when(i == 0)
    def _():
        m_scr[...] = jnp.full(m_scr.shape, -jnp.inf, jnp.float32)
        s_scr[...] = jnp.zeros(s_scr.shape, jnp.float32)

    m_prev = m_scr[...]
    m_new = jnp.maximum(m_prev, jnp.max(logits, axis=0, keepdims=True))
    e = jnp.exp(logits - m_new)
    bsum = lax.dot_general(
        jnp.ones((1, BV), jnp.float32), e, (((1,), (0,)), ((), ())),
        preferred_element_type=jnp.float32)
    s_scr[...] = s_scr[...] * jnp.exp(m_prev - m_new) + bsum
    m_scr[...] = m_new

    @pl.when(i == NV - 1)
    def _():
        m_out[...] = m_scr[...]
        s_out[...] = s_scr[...]


def _out_body(st_ref, w_ref, c_ref, o_ref):
    logits = _block_logits_t(st_ref, w_ref)
    o_ref[...] = jnp.exp(logits - c_ref[...])


def _softmax_stats(st_aug, wt_aug):
    return pl.pallas_call(
        _stats_body,
        grid=(NV,),
        in_specs=[
            pl.BlockSpec((D + 1, B), lambda i: (0, 0)),
            pl.BlockSpec((D + 1, BV), lambda i: (0, i)),
        ],
        out_specs=[
            pl.BlockSpec((1, B), lambda i: (0, 0)),
            pl.BlockSpec((1, B), lambda i: (0, 0)),
        ],
        out_shape=[
            jax.ShapeDtypeStruct((1, B), jnp.float32),
            jax.ShapeDtypeStruct((1, B), jnp.float32),
        ],
        scratch_shapes=[
            pltpu.VMEM((1, B), jnp.float32),
            pltpu.VMEM((1, B), jnp.float32),
        ],
        compiler_params=pltpu.CompilerParams(
            dimension_semantics=("arbitrary",)),
    )(st_aug, wt_aug)


def _softmax_write(st_aug, wt_aug, c):
    return pl.pallas_call(
        _out_body,
        grid=(NV,),
        in_specs=[
            pl.BlockSpec((D + 1, B), lambda i: (0, 0)),
            pl.BlockSpec((D + 1, BV), lambda i: (0, i)),
            pl.BlockSpec((1, B), lambda i: (0, 0)),
        ],
        out_specs=pl.BlockSpec((BV, B), lambda i: (i, 0)),
        out_shape=jax.ShapeDtypeStruct((V, B), jnp.float32),
        compiler_params=pltpu.CompilerParams(
            dimension_semantics=("arbitrary",)),
    )(st_aug, wt_aug, c)


def kernel(inputs, table, W, b):
    nc, ns = _sc_info()
    summed_t = _make_embed_sum(nc, ns)(inputs.T, table.T)
    st_aug = jnp.concatenate([summed_t, jnp.ones((1, B), jnp.float32)], axis=0)
    wt_pad = jnp.pad(W.T, ((0, 0), (0, VP - V)))
    b_pad = jnp.pad(b, (0, VP - V), constant_values=-1e30)
    wt_aug = jnp.concatenate([wt_pad, b_pad[None, :]], axis=0)
    m, s = _softmax_stats(st_aug, wt_aug)
    c = m + jnp.log(s)
    out_t = _softmax_write(st_aug, wt_aug, c)
    return out_t.T
